# baseline (speedup 1.0000x reference)
import jax
import jax.numpy as jnp
from jax.experimental import pallas as pl
from jax.experimental.pallas import tpu as pltpu

N = 320000
D = 128
CP = 128
B = 6400
NB = N // B


def _onehot_bf16(lab_bf, n_rows):
    ids = jax.lax.broadcasted_iota(jnp.int32, (1, CP), 1).astype(jnp.bfloat16)
    return jnp.where(lab_bf == ids, jnp.bfloat16(1), jnp.bfloat16(0))


def _row_sum_col(x_bf, ones_col):
    return jax.lax.dot_general(
        x_bf, ones_col, dimension_numbers=(((1,), (0,)), ((), ())),
        preferred_element_type=jnp.float32)


def _pass1_body(f_ref, old_ref, g_ref, lab_ref, ms_ref, gn2_ref, csum_ref,
                ccnt_ref):
    i = pl.program_id(0)
    f = f_ref[...]
    old = old_ref[...]
    g = g_ref[...]
    lab = lab_ref[...]

    ones_col = jnp.ones((D, 1), dtype=jnp.bfloat16)
    diff = f - old
    ms_ref[...] = _row_sum_col((diff * diff).astype(jnp.bfloat16), ones_col)
    gn2_ref[...] = _row_sum_col((g * g).astype(jnp.bfloat16), ones_col)

    oh_bf = _onehot_bf16(lab, B)
    csum_p = jax.lax.dot_general(
        oh_bf, f.astype(jnp.bfloat16),
        dimension_numbers=(((0,), (0,)), ((), ())),
        preferred_element_type=jnp.float32)
    ones_b = jnp.ones((B, 1), dtype=jnp.bfloat16)
    ccnt_p = jax.lax.dot_general(
        oh_bf, ones_b,
        dimension_numbers=(((0,), (0,)), ((), ())),
        preferred_element_type=jnp.float32)

    @pl.when(i == 0)
    def _():
        csum_ref[...] = jnp.zeros_like(csum_ref)
        ccnt_ref[...] = jnp.zeros_like(ccnt_ref)

    csum_ref[...] += csum_p
    ccnt_ref[...] += ccnt_p


def _pass2_body(f_ref, lab_ref, csum_ref, ccnt_ref, dist2_ref, cnt_ref):
    f = f_ref[...]
    lab = lab_ref[...]
    cnt = ccnt_ref[...]
    centroids = csum_ref[...] / jnp.maximum(cnt, 1.0)

    oh_bf = _onehot_bf16(lab, B)
    c_rows = jax.lax.dot_general(
        oh_bf, centroids.astype(jnp.bfloat16),
        dimension_numbers=(((1,), (0,)), ((), ())),
        preferred_element_type=jnp.float32)
    cnt_ref[...] = jax.lax.dot_general(
        oh_bf, cnt.astype(jnp.bfloat16),
        dimension_numbers=(((1,), (0,)), ((), ())),
        preferred_element_type=jnp.float32)

    ones_col = jnp.ones((D, 1), dtype=jnp.bfloat16)
    dd = f - c_rows
    dist2_ref[...] = _row_sum_col((dd * dd).astype(jnp.bfloat16), ones_col)


def _pass3_body(ms_ref, gn2_ref, dist2_ref, cnt_ref, comb_ref, p_ref):
    ls = 1.0 / (1.0 + ms_ref[...] * (1.0 / D))
    gs = 1.0 / (1.0 + jnp.exp(-jnp.sqrt(gn2_ref[...])))
    dist = jnp.sqrt(dist2_ref[...])
    typ = jnp.where(cnt_ref[...] > 1.0, 1.0 / (1.0 + dist), 1.0)
    comb = 0.3 * ls + 0.5 * gs + 0.2 * typ
    comb_ref[...] = comb
    m = jnp.max(comb)
    e = jnp.exp(comb - m)
    p_ref[...] = e * (1.0 / jnp.sum(e))


def kernel(features, labels, gradients, old_features):
    lab2d = labels.astype(jnp.int32).astype(jnp.bfloat16).reshape(N, 1)

    row_spec = pl.BlockSpec((B, D), lambda i: (i, 0))
    lab_spec = pl.BlockSpec((B, 1), lambda i: (i, 0))
    col_spec = pl.BlockSpec((B, 1), lambda i: (i, 0))
    acc_spec = pl.BlockSpec((CP, D), lambda i: (0, 0))
    cnt_spec = pl.BlockSpec((CP, 1), lambda i: (0, 0))
    col_shape = jax.ShapeDtypeStruct((N, 1), jnp.float32)

    ms, gn2, csum, ccnt = pl.pallas_call(
        _pass1_body,
        grid=(NB,),
        in_specs=[row_spec, row_spec, row_spec, lab_spec],
        out_specs=[col_spec, col_spec, acc_spec, cnt_spec],
        out_shape=[
            col_shape,
            col_shape,
            jax.ShapeDtypeStruct((CP, D), jnp.float32),
            jax.ShapeDtypeStruct((CP, 1), jnp.float32),
        ],
    )(features, old_features, gradients, lab2d)

    dist2, cnt_rows = pl.pallas_call(
        _pass2_body,
        grid=(NB,),
        in_specs=[row_spec, lab_spec, acc_spec, cnt_spec],
        out_specs=[col_spec, col_spec],
        out_shape=[col_shape, col_shape],
    )(features, lab2d, csum, ccnt)

    packed = lambda a: a.reshape(N // 128, 128)
    comb2d, probs = pl.pallas_call(
        _pass3_body,
        out_shape=[
            jax.ShapeDtypeStruct((N // 128, 128), jnp.float32),
            jax.ShapeDtypeStruct((N // 128, 128), jnp.float32),
        ],
    )(packed(ms), packed(gn2), packed(dist2), packed(cnt_rows))

    return comb2d.reshape(N), probs.reshape(N)


# speedup vs baseline: 1.7221x; 1.7221x over previous
import jax
import jax.numpy as jnp
from jax import lax
from jax.experimental import pallas as pl
from jax.experimental.pallas import tpu as pltpu
from jax.experimental.pallas import tpu_sc as plsc

N = 320000
D = 128
CP = 128
B = 6400
NB = N // B
BL = B // 128
BIG = 1e9

SC_NC = 2
SC_NS = 16
SC_NW = SC_NC * SC_NS
RW = N // SC_NW
RCH = 400
NCH = RW // RCH


def _sc_gn2_body(g_hbm, out_hbm, buf0, buf1, outv, sem0, sem1):
    wid = lax.axis_index("s") * SC_NC + lax.axis_index("c")
    base = wid * RW
    bufs = (buf0, buf1)
    sems = (sem0, sem1)
    lane = lax.iota(jnp.int32, 16)

    def start(ci, b):
        pltpu.make_async_copy(
            g_hbm.at[pl.ds(base + ci * RCH, RCH), :], bufs[b], sems[b]).start()

    def wait(b):
        pltpu.make_async_copy(
            g_hbm.at[pl.ds(base, RCH), :], bufs[b], sems[b]).wait()

    def gat(x, idx):
        return x.at[idx].get(mode='promise_in_bounds')

    def merge(u, v, s):
        h = s // 2
        uf = u + gat(u, lane ^ h)
        vf = v + gat(v, lane ^ h)
        return jnp.where((lane & h) == 0, uf, gat(vf, lane ^ h))

    bitrev = (0, 8, 4, 12, 2, 10, 6, 14, 1, 9, 5, 13, 3, 11, 7, 15)

    def compute(ci, b):
        buf = bufs[b]

        def group(r16, _):
            vecs = []
            for p in range(16):
                r = r16 * 16 + bitrev[p]
                v = buf[r, pl.ds(0, 16)]
                acc = v * v
                for j in range(1, 8):
                    v = buf[r, pl.ds(j * 16, 16)]
                    acc = acc + v * v
                vecs.append(acc)
            s = 16
            while len(vecs) > 1:
                vecs = [merge(vecs[2 * i], vecs[2 * i + 1], s)
                        for i in range(len(vecs) // 2)]
                s //= 2
            outv[pl.ds(ci * RCH + r16 * 16, 16)] = vecs[0]
            return 0

        lax.fori_loop(0, RCH // 16, group, 0, unroll=False)

    start(0, 0)

    def chunk_pair(ci2, _):
        for b in range(2):
            ci = ci2 * 2 + b

            @pl.when(ci < NCH)
            def _():
                wait(b)

                @pl.when(ci + 1 < NCH)
                def _():
                    start(ci + 1, 1 - b)

                compute(ci, b)
        return 0

    lax.fori_loop(0, (NCH + 1) // 2, chunk_pair, 0, unroll=False)
    pltpu.sync_copy(outv, out_hbm.at[pl.ds(base, RW)])


def _sc_gn2_call(gradients):
    mesh = plsc.VectorSubcoreMesh(core_axis_name="c", subcore_axis_name="s")
    return pl.kernel(
        _sc_gn2_body,
        out_type=jax.ShapeDtypeStruct((N,), jnp.float32),
        mesh=mesh,
        scratch_types=[
            pltpu.VMEM((RCH, D), jnp.float32),
            pltpu.VMEM((RCH, D), jnp.float32),
            pltpu.VMEM((RW,), jnp.float32),
            pltpu.SemaphoreType.DMA,
            pltpu.SemaphoreType.DMA,
        ],
    )(gradients)


def _onehot_bf16(lab_bf, n_rows):
    ids = jax.lax.broadcasted_iota(jnp.int32, (1, CP), 1).astype(jnp.bfloat16)
    return jnp.where(lab_bf == ids, jnp.bfloat16(1), jnp.bfloat16(0))


def _row_sums_packed(x_bf, ones_col):
    col = jax.lax.dot_general(
        x_bf, ones_col, dimension_numbers=(((1,), (0,)), ((), ())),
        preferred_element_type=jnp.float32)
    return col.astype(jnp.bfloat16).reshape(BL, 128)


def _pass1_body(f_ref, old_ref, lab_ref, ms_ref, csum_ref, ccnt_ref):
    i = pl.program_id(0)
    f = f_ref[...]
    old = old_ref[...]
    lab = lab_ref[...]

    ones_col = jnp.ones((D, 1), dtype=jnp.bfloat16)
    diff = f - old
    ms_ref[...] = _row_sums_packed((diff * diff).astype(jnp.bfloat16),
                                   ones_col)[None]

    oh_bf = _onehot_bf16(lab, B)
    csum_p = jax.lax.dot_general(
        oh_bf, f.astype(jnp.bfloat16),
        dimension_numbers=(((0,), (0,)), ((), ())),
        preferred_element_type=jnp.float32)
    ones_b = jnp.ones((B, 1), dtype=jnp.bfloat16)
    ccnt_p = jax.lax.dot_general(
        oh_bf, ones_b,
        dimension_numbers=(((0,), (0,)), ((), ())),
        preferred_element_type=jnp.float32)

    @pl.when(i == 0)
    def _():
        csum_ref[...] = jnp.zeros_like(csum_ref)
        ccnt_ref[...] = jnp.zeros_like(ccnt_ref)

    csum_ref[...] += csum_p
    ccnt_ref[...] += ccnt_p


def _pass2_body(f_ref, lab_ref, csum_ref, ccnt_ref, dm_ref):
    f = f_ref[...]
    lab = lab_ref[...]
    cnt = ccnt_ref[...]
    inv = 1.0 / jnp.maximum(cnt, 1.0)
    centroids = csum_ref[...] * inv
    pen = jnp.where(cnt > 1.0, 0.0, -BIG).astype(jnp.bfloat16)

    oh_bf = _onehot_bf16(lab, B)
    c_rows = jax.lax.dot_general(
        oh_bf, centroids.astype(jnp.bfloat16),
        dimension_numbers=(((1,), (0,)), ((), ())),
        preferred_element_type=jnp.float32)
    pen_col = jax.lax.dot_general(
        oh_bf, pen,
        dimension_numbers=(((1,), (0,)), ((), ())),
        preferred_element_type=jnp.float32)

    ones_col = jnp.ones((D, 1), dtype=jnp.bfloat16)
    dd = f - c_rows
    dist2_col = jax.lax.dot_general(
        (dd * dd).astype(jnp.bfloat16), ones_col,
        dimension_numbers=(((1,), (0,)), ((), ())),
        preferred_element_type=jnp.float32)
    dm_ref[...] = ((dist2_col + pen_col)
                   .astype(jnp.bfloat16).reshape(BL, 128)[None])


def _pass3_body(ms_ref, gn2_ref, dm_ref, comb_ref, p_ref):
    ls = 1.0 / (1.0 + ms_ref[...].astype(jnp.float32) * (1.0 / D))
    gs = 1.0 / (1.0 + jnp.exp(-jnp.sqrt(gn2_ref[...])))
    dm = dm_ref[...].astype(jnp.float32)
    typ = jnp.where(dm < 0.0, 1.0, 1.0 / (1.0 + jnp.sqrt(jnp.abs(dm))))
    comb = 0.3 * ls + 0.5 * gs + 0.2 * typ
    comb_ref[...] = comb
    m = jnp.max(comb)
    e = jnp.exp(comb - m)
    p_ref[...] = e * (1.0 / jnp.sum(e))


def kernel(features, labels, gradients, old_features):
    lab2d = labels.astype(jnp.int32).astype(jnp.bfloat16).reshape(N, 1)

    gn2_flat = _sc_gn2_call(gradients)

    row_spec = pl.BlockSpec((B, D), lambda i: (i, 0))
    lab_spec = pl.BlockSpec((B, 1), lambda i: (i, 0))
    packed_spec = pl.BlockSpec((1, BL, 128), lambda i: (i, 0, 0))
    acc_spec = pl.BlockSpec((CP, D), lambda i: (0, 0))
    cnt_spec = pl.BlockSpec((CP, 1), lambda i: (0, 0))
    packed_shape = jax.ShapeDtypeStruct((NB, BL, 128), jnp.bfloat16)

    ms, csum, ccnt = pl.pallas_call(
        _pass1_body,
        grid=(NB,),
        in_specs=[row_spec, row_spec, lab_spec],
        out_specs=[packed_spec, acc_spec, cnt_spec],
        out_shape=[
            packed_shape,
            jax.ShapeDtypeStruct((CP, D), jnp.float32),
            jax.ShapeDtypeStruct((CP, 1), jnp.float32),
        ],
    )(features, old_features, lab2d)

    dm = pl.pallas_call(
        _pass2_body,
        grid=(NB,),
        in_specs=[row_spec, lab_spec, acc_spec, cnt_spec],
        out_specs=packed_spec,
        out_shape=packed_shape,
    )(features, lab2d, csum, ccnt)

    packed = lambda a: a.reshape(N // 128, 128)
    comb2d, probs = pl.pallas_call(
        _pass3_body,
        out_shape=[
            jax.ShapeDtypeStruct((N // 128, 128), jnp.float32),
            jax.ShapeDtypeStruct((N // 128, 128), jnp.float32),
        ],
    )(packed(ms), packed(gn2_flat), packed(dm))

    return comb2d.reshape(N), probs.reshape(N)


# speedup vs baseline: 1.7925x; 1.0409x over previous
import jax
import jax.numpy as jnp
from jax import lax
from jax.experimental import pallas as pl
from jax.experimental.pallas import tpu as pltpu
from jax.experimental.pallas import tpu_sc as plsc

N = 320000
D = 128
CP = 128
B = 12800
NB = N // B
BL = B // 128
BIG = 1e9

SC_NC = 2
SC_NS = 16
SC_NW = SC_NC * SC_NS
RW = N // SC_NW
RCH = 400
NCH = RW // RCH


def _sc_gn2_body(g_hbm, out_hbm, buf0, buf1, outv, sem0, sem1):
    wid = lax.axis_index("s") * SC_NC + lax.axis_index("c")
    base = wid * RW
    bufs = (buf0, buf1)
    sems = (sem0, sem1)
    lane = lax.iota(jnp.int32, 16)

    def start(ci, b):
        pltpu.make_async_copy(
            g_hbm.at[pl.ds(base + ci * RCH, RCH), :], bufs[b], sems[b]).start()

    def wait(b):
        pltpu.make_async_copy(
            g_hbm.at[pl.ds(base, RCH), :], bufs[b], sems[b]).wait()

    def gat(x, idx):
        return x.at[idx].get(mode='promise_in_bounds')

    def merge(u, v, s):
        h = s // 2
        uf = u + gat(u, lane ^ h)
        vf = v + gat(v, lane ^ h)
        return jnp.where((lane & h) == 0, uf, gat(vf, lane ^ h))

    bitrev = (0, 8, 4, 12, 2, 10, 6, 14, 1, 9, 5, 13, 3, 11, 7, 15)

    def compute(ci, b):
        buf = bufs[b]

        def group(r16, _):
            vecs = []
            for p in range(16):
                r = r16 * 16 + bitrev[p]
                v = buf[r, pl.ds(0, 16)]
                acc = v * v
                for j in range(1, 8):
                    v = buf[r, pl.ds(j * 16, 16)]
                    acc = acc + v * v
                vecs.append(acc)
            s = 16
            while len(vecs) > 1:
                vecs = [merge(vecs[2 * i], vecs[2 * i + 1], s)
                        for i in range(len(vecs) // 2)]
                s //= 2
            outv[pl.ds(ci * RCH + r16 * 16, 16)] = vecs[0]
            return 0

        lax.fori_loop(0, RCH // 16, group, 0, unroll=False)

    start(0, 0)

    def chunk_pair(ci2, _):
        for b in range(2):
            ci = ci2 * 2 + b

            @pl.when(ci < NCH)
            def _():
                wait(b)

                @pl.when(ci + 1 < NCH)
                def _():
                    start(ci + 1, 1 - b)

                compute(ci, b)
        return 0

    lax.fori_loop(0, (NCH + 1) // 2, chunk_pair, 0, unroll=False)
    pltpu.sync_copy(outv, out_hbm.at[pl.ds(base, RW)])


def _sc_gn2_call(gradients):
    mesh = plsc.VectorSubcoreMesh(core_axis_name="c", subcore_axis_name="s")
    return pl.kernel(
        _sc_gn2_body,
        out_type=jax.ShapeDtypeStruct((N,), jnp.float32),
        mesh=mesh,
        scratch_types=[
            pltpu.VMEM((RCH, D), jnp.float32),
            pltpu.VMEM((RCH, D), jnp.float32),
            pltpu.VMEM((RW,), jnp.float32),
            pltpu.SemaphoreType.DMA,
            pltpu.SemaphoreType.DMA,
        ],
    )(gradients)


def _onehot_bf16(lab_bf, n_rows):
    ids = jax.lax.broadcasted_iota(jnp.int32, (1, CP), 1).astype(jnp.bfloat16)
    return jnp.where(lab_bf == ids, jnp.bfloat16(1), jnp.bfloat16(0))


def _row_sums_packed(x_bf, ones_col):
    col = jax.lax.dot_general(
        x_bf, ones_col, dimension_numbers=(((1,), (0,)), ((), ())),
        preferred_element_type=jnp.float32)
    return col.astype(jnp.bfloat16).reshape(BL, 128)


def _pass1_body(f_ref, old_ref, lab_ref, ms_ref, csum_ref, ccnt_ref):
    i = pl.program_id(0)
    f = f_ref[...]
    old = old_ref[...]
    lab = lab_ref[...]

    ones_col = jnp.ones((D, 1), dtype=jnp.bfloat16)
    diff = f - old
    ms_ref[...] = _row_sums_packed((diff * diff).astype(jnp.bfloat16),
                                   ones_col)[None]

    oh_bf = _onehot_bf16(lab, B)
    csum_p = jax.lax.dot_general(
        oh_bf, f.astype(jnp.bfloat16),
        dimension_numbers=(((0,), (0,)), ((), ())),
        preferred_element_type=jnp.float32)
    ones_b = jnp.ones((B, 1), dtype=jnp.bfloat16)
    ccnt_p = jax.lax.dot_general(
        oh_bf, ones_b,
        dimension_numbers=(((0,), (0,)), ((), ())),
        preferred_element_type=jnp.float32)

    @pl.when(i == 0)
    def _():
        csum_ref[...] = jnp.zeros_like(csum_ref)
        ccnt_ref[...] = jnp.zeros_like(ccnt_ref)

    csum_ref[...] += csum_p
    ccnt_ref[...] += ccnt_p


def _pass2_body(f_ref, lab_ref, csum_ref, ccnt_ref, dm_ref):
    f = f_ref[...]
    lab = lab_ref[...]
    cnt = ccnt_ref[...]
    inv = 1.0 / jnp.maximum(cnt, 1.0)
    centroids = csum_ref[...] * inv
    pen = jnp.where(cnt > 1.0, 0.0, -BIG).astype(jnp.bfloat16)

    oh_bf = _onehot_bf16(lab, B)
    c_rows = jax.lax.dot_general(
        oh_bf, centroids.astype(jnp.bfloat16),
        dimension_numbers=(((1,), (0,)), ((), ())),
        preferred_element_type=jnp.float32)
    pen_col = jax.lax.dot_general(
        oh_bf, pen,
        dimension_numbers=(((1,), (0,)), ((), ())),
        preferred_element_type=jnp.float32)

    ones_col = jnp.ones((D, 1), dtype=jnp.bfloat16)
    dd = f - c_rows
    dist2_col = jax.lax.dot_general(
        (dd * dd).astype(jnp.bfloat16), ones_col,
        dimension_numbers=(((1,), (0,)), ((), ())),
        preferred_element_type=jnp.float32)
    dm_ref[...] = ((dist2_col + pen_col)
                   .astype(jnp.bfloat16).reshape(BL, 128)[None])


def _pass3_body(ms_ref, gn2_ref, dm_ref, comb_ref, p_ref):
    ls = 1.0 / (1.0 + ms_ref[...].astype(jnp.float32) * (1.0 / D))
    gs = 1.0 / (1.0 + jnp.exp(-jnp.sqrt(gn2_ref[...])))
    dm = dm_ref[...].astype(jnp.float32)
    typ = jnp.where(dm < 0.0, 1.0, 1.0 / (1.0 + jnp.sqrt(jnp.abs(dm))))
    comb = 0.3 * ls + 0.5 * gs + 0.2 * typ
    comb_ref[...] = comb
    m = jnp.max(comb)
    e = jnp.exp(comb - m)
    p_ref[...] = e * (1.0 / jnp.sum(e))


def kernel(features, labels, gradients, old_features):
    lab2d = labels.astype(jnp.int32).astype(jnp.bfloat16).reshape(N, 1)

    gn2_flat = _sc_gn2_call(gradients)

    row_spec = pl.BlockSpec((B, D), lambda i: (i, 0))
    lab_spec = pl.BlockSpec((B, 1), lambda i: (i, 0))
    packed_spec = pl.BlockSpec((1, BL, 128), lambda i: (i, 0, 0))
    acc_spec = pl.BlockSpec((CP, D), lambda i: (0, 0))
    cnt_spec = pl.BlockSpec((CP, 1), lambda i: (0, 0))
    packed_shape = jax.ShapeDtypeStruct((NB, BL, 128), jnp.bfloat16)

    ms, csum, ccnt = pl.pallas_call(
        _pass1_body,
        grid=(NB,),
        in_specs=[row_spec, row_spec, lab_spec],
        out_specs=[packed_spec, acc_spec, cnt_spec],
        out_shape=[
            packed_shape,
            jax.ShapeDtypeStruct((CP, D), jnp.float32),
            jax.ShapeDtypeStruct((CP, 1), jnp.float32),
        ],
    )(features, old_features, lab2d)

    dm = pl.pallas_call(
        _pass2_body,
        grid=(NB,),
        in_specs=[row_spec, lab_spec, acc_spec, cnt_spec],
        out_specs=packed_spec,
        out_shape=packed_shape,
    )(features, lab2d, csum, ccnt)

    packed = lambda a: a.reshape(N // 128, 128)
    comb2d, probs = pl.pallas_call(
        _pass3_body,
        out_shape=[
            jax.ShapeDtypeStruct((N // 128, 128), jnp.float32),
            jax.ShapeDtypeStruct((N // 128, 128), jnp.float32),
        ],
    )(packed(ms), packed(gn2_flat), packed(dm))

    return comb2d.reshape(N), probs.reshape(N)


# speedup vs baseline: 1.8031x; 1.0059x over previous
import jax
import jax.numpy as jnp
from jax import lax
from jax.experimental import pallas as pl
from jax.experimental.pallas import tpu as pltpu
from jax.experimental.pallas import tpu_sc as plsc

N = 320000
D = 128
CP = 128
B = 16000
NB = N // B
BL = B // 128
BIG = 1e9

SC_NC = 2
SC_NS = 16
SC_NW = SC_NC * SC_NS
RW = N // SC_NW
RCH = 400
NCH = RW // RCH


def _sc_gn2_body(g_hbm, out_hbm, buf0, buf1, outv, sem0, sem1):
    wid = lax.axis_index("s") * SC_NC + lax.axis_index("c")
    base = wid * RW
    bufs = (buf0, buf1)
    sems = (sem0, sem1)
    lane = lax.iota(jnp.int32, 16)

    def start(ci, b):
        pltpu.make_async_copy(
            g_hbm.at[pl.ds(base + ci * RCH, RCH), :], bufs[b], sems[b]).start()

    def wait(b):
        pltpu.make_async_copy(
            g_hbm.at[pl.ds(base, RCH), :], bufs[b], sems[b]).wait()

    def gat(x, idx):
        return x.at[idx].get(mode='promise_in_bounds')

    def merge(u, v, s):
        h = s // 2
        uf = u + gat(u, lane ^ h)
        vf = v + gat(v, lane ^ h)
        return jnp.where((lane & h) == 0, uf, gat(vf, lane ^ h))

    bitrev = (0, 8, 4, 12, 2, 10, 6, 14, 1, 9, 5, 13, 3, 11, 7, 15)

    def compute(ci, b):
        buf = bufs[b]

        def group(r16, _):
            vecs = []
            for p in range(16):
                r = r16 * 16 + bitrev[p]
                v = buf[r, pl.ds(0, 16)]
                acc = v * v
                for j in range(1, 8):
                    v = buf[r, pl.ds(j * 16, 16)]
                    acc = acc + v * v
                vecs.append(acc)
            s = 16
            while len(vecs) > 1:
                vecs = [merge(vecs[2 * i], vecs[2 * i + 1], s)
                        for i in range(len(vecs) // 2)]
                s //= 2
            outv[pl.ds(ci * RCH + r16 * 16, 16)] = vecs[0]
            return 0

        lax.fori_loop(0, RCH // 16, group, 0, unroll=False)

    start(0, 0)

    def chunk_pair(ci2, _):
        for b in range(2):
            ci = ci2 * 2 + b

            @pl.when(ci < NCH)
            def _():
                wait(b)

                @pl.when(ci + 1 < NCH)
                def _():
                    start(ci + 1, 1 - b)

                compute(ci, b)
        return 0

    lax.fori_loop(0, (NCH + 1) // 2, chunk_pair, 0, unroll=False)
    pltpu.sync_copy(outv, out_hbm.at[pl.ds(base, RW)])


def _sc_gn2_call(gradients):
    mesh = plsc.VectorSubcoreMesh(core_axis_name="c", subcore_axis_name="s")
    return pl.kernel(
        _sc_gn2_body,
        out_type=jax.ShapeDtypeStruct((N,), jnp.float32),
        mesh=mesh,
        scratch_types=[
            pltpu.VMEM((RCH, D), jnp.float32),
            pltpu.VMEM((RCH, D), jnp.float32),
            pltpu.VMEM((RW,), jnp.float32),
            pltpu.SemaphoreType.DMA,
            pltpu.SemaphoreType.DMA,
        ],
    )(gradients)


def _onehot_bf16(lab_bf, n_rows):
    ids = jax.lax.broadcasted_iota(jnp.int32, (1, CP), 1).astype(jnp.bfloat16)
    return jnp.where(lab_bf == ids, jnp.bfloat16(1), jnp.bfloat16(0))


def _row_sums_packed(x_bf, ones_col):
    col = jax.lax.dot_general(
        x_bf, ones_col, dimension_numbers=(((1,), (0,)), ((), ())),
        preferred_element_type=jnp.float32)
    return col.astype(jnp.bfloat16).reshape(BL, 128)


def _pass1_body(f_ref, old_ref, lab_ref, ms_ref, csum_ref, ccnt_ref):
    i = pl.program_id(0)
    f = f_ref[...]
    old = old_ref[...]
    lab = lab_ref[...]

    ones_col = jnp.ones((D, 1), dtype=jnp.bfloat16)
    diff = f - old
    ms_ref[...] = _row_sums_packed((diff * diff).astype(jnp.bfloat16),
                                   ones_col)[None]

    oh_bf = _onehot_bf16(lab, B)
    csum_p = jax.lax.dot_general(
        oh_bf, f.astype(jnp.bfloat16),
        dimension_numbers=(((0,), (0,)), ((), ())),
        preferred_element_type=jnp.float32)
    ones_b = jnp.ones((B, 1), dtype=jnp.bfloat16)
    ccnt_p = jax.lax.dot_general(
        oh_bf, ones_b,
        dimension_numbers=(((0,), (0,)), ((), ())),
        preferred_element_type=jnp.float32)

    @pl.when(i == 0)
    def _():
        csum_ref[...] = jnp.zeros_like(csum_ref)
        ccnt_ref[...] = jnp.zeros_like(ccnt_ref)

    csum_ref[...] += csum_p
    ccnt_ref[...] += ccnt_p


def _pass2_body(f_ref, lab_ref, csum_ref, ccnt_ref, dm_ref):
    f = f_ref[...]
    lab = lab_ref[...]
    cnt = ccnt_ref[...]
    inv = 1.0 / jnp.maximum(cnt, 1.0)
    centroids = csum_ref[...] * inv
    pen = jnp.where(cnt > 1.0, 0.0, -BIG).astype(jnp.bfloat16)

    oh_bf = _onehot_bf16(lab, B)
    c_rows = jax.lax.dot_general(
        oh_bf, centroids.astype(jnp.bfloat16),
        dimension_numbers=(((1,), (0,)), ((), ())),
        preferred_element_type=jnp.float32)
    pen_col = jax.lax.dot_general(
        oh_bf, pen,
        dimension_numbers=(((1,), (0,)), ((), ())),
        preferred_element_type=jnp.float32)

    ones_col = jnp.ones((D, 1), dtype=jnp.bfloat16)
    dd = f - c_rows
    dist2_col = jax.lax.dot_general(
        (dd * dd).astype(jnp.bfloat16), ones_col,
        dimension_numbers=(((1,), (0,)), ((), ())),
        preferred_element_type=jnp.float32)
    dm_ref[...] = ((dist2_col + pen_col)
                   .astype(jnp.bfloat16).reshape(BL, 128)[None])


def _pass3_body(ms_ref, gn2_ref, dm_ref, comb_ref, p_ref):
    ls = 1.0 / (1.0 + ms_ref[...].astype(jnp.float32) * (1.0 / D))
    gs = 1.0 / (1.0 + jnp.exp(-jnp.sqrt(gn2_ref[...])))
    dm = dm_ref[...].astype(jnp.float32)
    typ = jnp.where(dm < 0.0, 1.0, 1.0 / (1.0 + jnp.sqrt(jnp.abs(dm))))
    comb = 0.3 * ls + 0.5 * gs + 0.2 * typ
    comb_ref[...] = comb
    m = jnp.max(comb)
    e = jnp.exp(comb - m)
    p_ref[...] = e * (1.0 / jnp.sum(e))


def kernel(features, labels, gradients, old_features):
    lab2d = labels.astype(jnp.int32).astype(jnp.bfloat16).reshape(N, 1)

    gn2_flat = _sc_gn2_call(gradients)

    row_spec = pl.BlockSpec((B, D), lambda i: (i, 0))
    lab_spec = pl.BlockSpec((B, 1), lambda i: (i, 0))
    packed_spec = pl.BlockSpec((1, BL, 128), lambda i: (i, 0, 0))
    acc_spec = pl.BlockSpec((CP, D), lambda i: (0, 0))
    cnt_spec = pl.BlockSpec((CP, 1), lambda i: (0, 0))
    packed_shape = jax.ShapeDtypeStruct((NB, BL, 128), jnp.bfloat16)

    ms, csum, ccnt = pl.pallas_call(
        _pass1_body,
        grid=(NB,),
        in_specs=[row_spec, row_spec, lab_spec],
        out_specs=[packed_spec, acc_spec, cnt_spec],
        out_shape=[
            packed_shape,
            jax.ShapeDtypeStruct((CP, D), jnp.float32),
            jax.ShapeDtypeStruct((CP, 1), jnp.float32),
        ],
    )(features, old_features, lab2d)

    dm = pl.pallas_call(
        _pass2_body,
        grid=(NB,),
        in_specs=[row_spec, lab_spec, acc_spec, cnt_spec],
        out_specs=packed_spec,
        out_shape=packed_shape,
    )(features, lab2d, csum, ccnt)

    packed = lambda a: a.reshape(N // 128, 128)
    comb2d, probs = pl.pallas_call(
        _pass3_body,
        out_shape=[
            jax.ShapeDtypeStruct((N // 128, 128), jnp.float32),
            jax.ShapeDtypeStruct((N // 128, 128), jnp.float32),
        ],
    )(packed(ms), packed(gn2_flat), packed(dm))

    return comb2d.reshape(N), probs.reshape(N)


# speedup vs baseline: 1.8610x; 1.0322x over previous
import jax
import jax.numpy as jnp
from jax import lax
from jax.experimental import pallas as pl
from jax.experimental.pallas import tpu as pltpu
from jax.experimental.pallas import tpu_sc as plsc

N = 320000
D = 128
CP = 128
B = 16000
NB = N // B
BL = B // 128
BIG = 1e9

SC_NC = 2
SC_NS = 16
SC_NW = SC_NC * SC_NS
RW = N // SC_NW
RCH = 80
NCH = RW // RCH


def _sc_body(g_hbm, f_hbm, old_hbm, gn2_hbm, ms_hbm,
             gb0, gb1, fb0, fb1, ob0, ob1, gn2v, msv, sem0, sem1):
    wid = lax.axis_index("s") * SC_NC + lax.axis_index("c")
    base = wid * RW
    gbufs = (gb0, gb1)
    fbufs = (fb0, fb1)
    obufs = (ob0, ob1)
    sems = (sem0, sem1)
    lane = lax.iota(jnp.int32, 16)

    def start(ci, b):
        src = pl.ds(base + ci * RCH, RCH)
        pltpu.make_async_copy(g_hbm.at[src, :], gbufs[b], sems[b]).start()
        pltpu.make_async_copy(f_hbm.at[src, :], fbufs[b], sems[b]).start()
        pltpu.make_async_copy(old_hbm.at[src, :], obufs[b], sems[b]).start()

    def wait(b):
        src = pl.ds(base, RCH)
        pltpu.make_async_copy(g_hbm.at[src, :], gbufs[b], sems[b]).wait()
        pltpu.make_async_copy(f_hbm.at[src, :], fbufs[b], sems[b]).wait()
        pltpu.make_async_copy(old_hbm.at[src, :], obufs[b], sems[b]).wait()

    def gat(x, idx):
        return x.at[idx].get(mode='promise_in_bounds')

    def merge(u, v, s):
        h = s // 2
        uf = u + gat(u, lane ^ h)
        vf = v + gat(v, lane ^ h)
        return jnp.where((lane & h) == 0, uf, gat(vf, lane ^ h))

    def tree(vecs):
        s = 16
        while len(vecs) > 1:
            vecs = [merge(vecs[2 * i], vecs[2 * i + 1], s)
                    for i in range(len(vecs) // 2)]
            s //= 2
        return vecs[0]

    bitrev = (0, 8, 4, 12, 2, 10, 6, 14, 1, 9, 5, 13, 3, 11, 7, 15)

    def compute(ci, b):
        gb, fb, ob = gbufs[b], fbufs[b], obufs[b]

        def group(r16, _):
            gvecs = []
            mvecs = []
            for p in range(16):
                r = r16 * 16 + bitrev[p]
                v = gb[r, pl.ds(0, 16)]
                gacc = v * v
                d = fb[r, pl.ds(0, 16)] - ob[r, pl.ds(0, 16)]
                macc = d * d
                for j in range(1, 8):
                    sl = pl.ds(j * 16, 16)
                    v = gb[r, sl]
                    gacc = gacc + v * v
                    d = fb[r, sl] - ob[r, sl]
                    macc = macc + d * d
                gvecs.append(gacc)
                mvecs.append(macc)
            off = pl.ds(ci * RCH + r16 * 16, 16)
            gn2v[off] = tree(gvecs)
            msv[off] = tree(mvecs)
            return 0

        lax.fori_loop(0, RCH // 16, group, 0, unroll=False)

    start(0, 0)

    def chunk_pair(ci2, _):
        for b in range(2):
            ci = ci2 * 2 + b

            @pl.when(ci < NCH)
            def _():
                wait(b)

                @pl.when(ci + 1 < NCH)
                def _():
                    start(ci + 1, 1 - b)

                compute(ci, b)
        return 0

    lax.fori_loop(0, (NCH + 1) // 2, chunk_pair, 0, unroll=False)
    pltpu.sync_copy(gn2v, gn2_hbm.at[pl.ds(base, RW)])
    pltpu.sync_copy(msv, ms_hbm.at[pl.ds(base, RW)])


def _sc_call(gradients, features, old_features):
    mesh = plsc.VectorSubcoreMesh(core_axis_name="c", subcore_axis_name="s")
    return pl.kernel(
        _sc_body,
        out_type=[
            jax.ShapeDtypeStruct((N,), jnp.float32),
            jax.ShapeDtypeStruct((N,), jnp.float32),
        ],
        mesh=mesh,
        scratch_types=[
            pltpu.VMEM((RCH, D), jnp.float32),
            pltpu.VMEM((RCH, D), jnp.float32),
            pltpu.VMEM((RCH, D), jnp.float32),
            pltpu.VMEM((RCH, D), jnp.float32),
            pltpu.VMEM((RCH, D), jnp.float32),
            pltpu.VMEM((RCH, D), jnp.float32),
            pltpu.VMEM((RW,), jnp.float32),
            pltpu.VMEM((RW,), jnp.float32),
            pltpu.SemaphoreType.DMA,
            pltpu.SemaphoreType.DMA,
        ],
    )(gradients, features, old_features)


def _onehot_bf16(lab_bf, n_rows):
    ids = jax.lax.broadcasted_iota(jnp.int32, (1, CP), 1).astype(jnp.bfloat16)
    return jnp.where(lab_bf == ids, jnp.bfloat16(1), jnp.bfloat16(0))


def _pass1_body(f_ref, lab_ref, csum_ref, ccnt_ref):
    i = pl.program_id(0)
    f = f_ref[...]
    lab = lab_ref[...]

    oh_bf = _onehot_bf16(lab, B)
    csum_p = jax.lax.dot_general(
        oh_bf, f.astype(jnp.bfloat16),
        dimension_numbers=(((0,), (0,)), ((), ())),
        preferred_element_type=jnp.float32)
    ones_b = jnp.ones((B, 1), dtype=jnp.bfloat16)
    ccnt_p = jax.lax.dot_general(
        oh_bf, ones_b,
        dimension_numbers=(((0,), (0,)), ((), ())),
        preferred_element_type=jnp.float32)

    @pl.when(i == 0)
    def _():
        csum_ref[...] = jnp.zeros_like(csum_ref)
        ccnt_ref[...] = jnp.zeros_like(ccnt_ref)

    csum_ref[...] += csum_p
    ccnt_ref[...] += ccnt_p


def _pass2_body(f_ref, lab_ref, csum_ref, ccnt_ref, dm_ref):
    f = f_ref[...]
    lab = lab_ref[...]
    cnt = ccnt_ref[...]
    inv = 1.0 / jnp.maximum(cnt, 1.0)
    centroids = csum_ref[...] * inv
    pen = jnp.where(cnt > 1.0, 0.0, -BIG).astype(jnp.bfloat16)

    oh_bf = _onehot_bf16(lab, B)
    c_rows = jax.lax.dot_general(
        oh_bf, centroids.astype(jnp.bfloat16),
        dimension_numbers=(((1,), (0,)), ((), ())),
        preferred_element_type=jnp.float32)
    pen_col = jax.lax.dot_general(
        oh_bf, pen,
        dimension_numbers=(((1,), (0,)), ((), ())),
        preferred_element_type=jnp.float32)

    ones_col = jnp.ones((D, 1), dtype=jnp.bfloat16)
    dd = f - c_rows
    dist2_col = jax.lax.dot_general(
        (dd * dd).astype(jnp.bfloat16), ones_col,
        dimension_numbers=(((1,), (0,)), ((), ())),
        preferred_element_type=jnp.float32)
    dm_ref[...] = ((dist2_col + pen_col)
                   .astype(jnp.bfloat16).reshape(BL, 128)[None])


def _pass3_body(ms_ref, gn2_ref, dm_ref, comb_ref, p_ref):
    ls = 1.0 / (1.0 + ms_ref[...] * (1.0 / D))
    gs = 1.0 / (1.0 + jnp.exp(-jnp.sqrt(gn2_ref[...])))
    dm = dm_ref[...].astype(jnp.float32)
    typ = jnp.where(dm < 0.0, 1.0, 1.0 / (1.0 + jnp.sqrt(jnp.abs(dm))))
    comb = 0.3 * ls + 0.5 * gs + 0.2 * typ
    comb_ref[...] = comb
    m = jnp.max(comb)
    e = jnp.exp(comb - m)
    p_ref[...] = e * (1.0 / jnp.sum(e))


def kernel(features, labels, gradients, old_features):
    lab2d = labels.astype(jnp.int32).astype(jnp.bfloat16).reshape(N, 1)

    gn2_flat, ms_flat = _sc_call(gradients, features, old_features)

    row_spec = pl.BlockSpec((B, D), lambda i: (i, 0))
    lab_spec = pl.BlockSpec((B, 1), lambda i: (i, 0))
    packed_spec = pl.BlockSpec((1, BL, 128), lambda i: (i, 0, 0))
    acc_spec = pl.BlockSpec((CP, D), lambda i: (0, 0))
    cnt_spec = pl.BlockSpec((CP, 1), lambda i: (0, 0))

    csum, ccnt = pl.pallas_call(
        _pass1_body,
        grid=(NB,),
        in_specs=[row_spec, lab_spec],
        out_specs=[acc_spec, cnt_spec],
        out_shape=[
            jax.ShapeDtypeStruct((CP, D), jnp.float32),
            jax.ShapeDtypeStruct((CP, 1), jnp.float32),
        ],
    )(features, lab2d)

    dm = pl.pallas_call(
        _pass2_body,
        grid=(NB,),
        in_specs=[row_spec, lab_spec, acc_spec, cnt_spec],
        out_specs=packed_spec,
        out_shape=jax.ShapeDtypeStruct((NB, BL, 128), jnp.bfloat16),
    )(features, lab2d, csum, ccnt)

    packed = lambda a: a.reshape(N // 128, 128)
    comb2d, probs = pl.pallas_call(
        _pass3_body,
        out_shape=[
            jax.ShapeDtypeStruct((N // 128, 128), jnp.float32),
            jax.ShapeDtypeStruct((N // 128, 128), jnp.float32),
        ],
    )(packed(ms_flat), packed(gn2_flat), packed(dm))

    return comb2d.reshape(N), probs.reshape(N)


# speedup vs baseline: 2.4083x; 1.2940x over previous
import jax
import jax.numpy as jnp
from jax import lax
from jax.experimental import pallas as pl
from jax.experimental.pallas import tpu as pltpu
from jax.experimental.pallas import tpu_sc as plsc

N = 320000
D = 128
CP = 128
B = 16000
NB = N // B
BL = B // 128
BIG = 1e9

SC_NC = 2
SC_NS = 16
SC_NW = SC_NC * SC_NS
RW = N // SC_NW
RCH = 80
NCH = RW // RCH


def _sc_body(g_hbm, f_hbm, old_hbm, gn2_hbm, ms_hbm,
             gb0, gb1, fb0, fb1, ob0, ob1, gn2v, msv, sem0, sem1):
    wid = lax.axis_index("s") * SC_NC + lax.axis_index("c")
    base = wid * RW
    gbufs = (gb0, gb1)
    fbufs = (fb0, fb1)
    obufs = (ob0, ob1)
    sems = (sem0, sem1)
    lane = lax.iota(jnp.int32, 16)

    def start(ci, b):
        src = pl.ds(base + ci * RCH, RCH)
        pltpu.make_async_copy(g_hbm.at[src, :], gbufs[b], sems[b]).start()
        pltpu.make_async_copy(f_hbm.at[src, :], fbufs[b], sems[b]).start()
        pltpu.make_async_copy(old_hbm.at[src, :], obufs[b], sems[b]).start()

    def wait(b):
        src = pl.ds(base, RCH)
        pltpu.make_async_copy(g_hbm.at[src, :], gbufs[b], sems[b]).wait()
        pltpu.make_async_copy(f_hbm.at[src, :], fbufs[b], sems[b]).wait()
        pltpu.make_async_copy(old_hbm.at[src, :], obufs[b], sems[b]).wait()

    def gat(x, idx):
        return x.at[idx].get(mode='promise_in_bounds')

    def merge(u, v, s):
        h = s // 2
        uf = u + gat(u, lane ^ h)
        vf = v + gat(v, lane ^ h)
        return jnp.where((lane & h) == 0, uf, gat(vf, lane ^ h))

    def tree(vecs):
        s = 16
        while len(vecs) > 1:
            vecs = [merge(vecs[2 * i], vecs[2 * i + 1], s)
                    for i in range(len(vecs) // 2)]
            s //= 2
        return vecs[0]

    bitrev = (0, 8, 4, 12, 2, 10, 6, 14, 1, 9, 5, 13, 3, 11, 7, 15)

    def compute(ci, b):
        gb, fb, ob = gbufs[b], fbufs[b], obufs[b]

        def group(r16, _):
            gvecs = []
            mvecs = []
            for p in range(16):
                r = r16 * 16 + bitrev[p]
                v = gb[r, pl.ds(0, 16)]
                gacc = v * v
                d = fb[r, pl.ds(0, 16)] - ob[r, pl.ds(0, 16)]
                macc = d * d
                for j in range(1, 8):
                    sl = pl.ds(j * 16, 16)
                    v = gb[r, sl]
                    gacc = gacc + v * v
                    d = fb[r, sl] - ob[r, sl]
                    macc = macc + d * d
                gvecs.append(gacc)
                mvecs.append(macc)
            off = pl.ds(ci * RCH + r16 * 16, 16)
            gn2v[off] = tree(gvecs)
            msv[off] = tree(mvecs)
            return 0

        lax.fori_loop(0, RCH // 16, group, 0, unroll=False)

    start(0, 0)

    def chunk_pair(ci2, _):
        for b in range(2):
            ci = ci2 * 2 + b

            @pl.when(ci < NCH)
            def _():
                wait(b)

                @pl.when(ci + 1 < NCH)
                def _():
                    start(ci + 1, 1 - b)

                compute(ci, b)
        return 0

    lax.fori_loop(0, (NCH + 1) // 2, chunk_pair, 0, unroll=False)
    pltpu.sync_copy(gn2v, gn2_hbm.at[pl.ds(base, RW)])
    pltpu.sync_copy(msv, ms_hbm.at[pl.ds(base, RW)])


def _sc_call(gradients, features, old_features):
    mesh = plsc.VectorSubcoreMesh(core_axis_name="c", subcore_axis_name="s")
    return pl.kernel(
        _sc_body,
        out_type=[
            jax.ShapeDtypeStruct((N,), jnp.float32),
            jax.ShapeDtypeStruct((N,), jnp.float32),
        ],
        mesh=mesh,
        scratch_types=[
            pltpu.VMEM((RCH, D), jnp.float32),
            pltpu.VMEM((RCH, D), jnp.float32),
            pltpu.VMEM((RCH, D), jnp.float32),
            pltpu.VMEM((RCH, D), jnp.float32),
            pltpu.VMEM((RCH, D), jnp.float32),
            pltpu.VMEM((RCH, D), jnp.float32),
            pltpu.VMEM((RW,), jnp.float32),
            pltpu.VMEM((RW,), jnp.float32),
            pltpu.SemaphoreType.DMA,
            pltpu.SemaphoreType.DMA,
        ],
    )(gradients, features, old_features)


def _onehot_t_bf16(lab_row):
    ids = jax.lax.broadcasted_iota(jnp.int32, (CP, 1), 0).astype(jnp.bfloat16)
    return jnp.where(lab_row == ids, jnp.bfloat16(1), jnp.bfloat16(0))


def _pass1_body(f_ref, lab_ref, csum_ref, ccnt_ref):
    i = pl.program_id(0)
    f = f_ref[...]
    lab = lab_ref[0]

    oht = _onehot_t_bf16(lab)
    csum_p = jax.lax.dot_general(
        oht, f.astype(jnp.bfloat16),
        dimension_numbers=(((1,), (0,)), ((), ())),
        preferred_element_type=jnp.float32)
    ones_b = jnp.ones((B, 1), dtype=jnp.bfloat16)
    ccnt_p = jax.lax.dot_general(
        oht, ones_b,
        dimension_numbers=(((1,), (0,)), ((), ())),
        preferred_element_type=jnp.float32)

    @pl.when(i == 0)
    def _():
        csum_ref[...] = jnp.zeros_like(csum_ref)
        ccnt_ref[...] = jnp.zeros_like(ccnt_ref)

    csum_ref[...] += csum_p
    ccnt_ref[...] += ccnt_p


def _pass2_body(f_ref, lab_ref, csum_ref, ccnt_ref, dm_ref):
    f = f_ref[...]
    lab = lab_ref[0]
    cnt = ccnt_ref[...]
    inv = 1.0 / jnp.maximum(cnt, 1.0)
    centroids = csum_ref[...] * inv
    pen = jnp.where(cnt > 1.0, 0.0, -BIG).astype(jnp.bfloat16)

    oht = _onehot_t_bf16(lab)
    c_rows = jax.lax.dot_general(
        oht, centroids.astype(jnp.bfloat16),
        dimension_numbers=(((0,), (0,)), ((), ())),
        preferred_element_type=jnp.float32)
    pen_col = jax.lax.dot_general(
        oht, pen,
        dimension_numbers=(((0,), (0,)), ((), ())),
        preferred_element_type=jnp.float32)

    ones_col = jnp.ones((D, 1), dtype=jnp.bfloat16)
    dd = f - c_rows
    dist2_col = jax.lax.dot_general(
        (dd * dd).astype(jnp.bfloat16), ones_col,
        dimension_numbers=(((1,), (0,)), ((), ())),
        preferred_element_type=jnp.float32)
    dm_ref[...] = ((dist2_col + pen_col)
                   .astype(jnp.bfloat16).reshape(BL, 128)[None])


def _pass3_body(ms_ref, gn2_ref, dm_ref, comb_ref, p_ref):
    ls = 1.0 / (1.0 + ms_ref[...] * (1.0 / D))
    gs = 1.0 / (1.0 + jnp.exp(-jnp.sqrt(gn2_ref[...])))
    dm = dm_ref[...].astype(jnp.float32)
    typ = jnp.where(dm < 0.0, 1.0, 1.0 / (1.0 + jnp.sqrt(jnp.abs(dm))))
    comb = 0.3 * ls + 0.5 * gs + 0.2 * typ
    comb_ref[...] = comb
    m = jnp.max(comb)
    e = jnp.exp(comb - m)
    p_ref[...] = e * (1.0 / jnp.sum(e))


def kernel(features, labels, gradients, old_features):
    lab3d = labels.astype(jnp.int32).astype(jnp.bfloat16).reshape(NB, 1, B)

    gn2_flat, ms_flat = _sc_call(gradients, features, old_features)

    row_spec = pl.BlockSpec((B, D), lambda i: (i, 0))
    lab_spec = pl.BlockSpec((1, 1, B), lambda i: (i, 0, 0))
    packed_spec = pl.BlockSpec((1, BL, 128), lambda i: (i, 0, 0))
    acc_spec = pl.BlockSpec((CP, D), lambda i: (0, 0))
    cnt_spec = pl.BlockSpec((CP, 1), lambda i: (0, 0))

    csum, ccnt = pl.pallas_call(
        _pass1_body,
        grid=(NB,),
        in_specs=[row_spec, lab_spec],
        out_specs=[acc_spec, cnt_spec],
        out_shape=[
            jax.ShapeDtypeStruct((CP, D), jnp.float32),
            jax.ShapeDtypeStruct((CP, 1), jnp.float32),
        ],
    )(features, lab3d)

    dm = pl.pallas_call(
        _pass2_body,
        grid=(NB,),
        in_specs=[row_spec, lab_spec, acc_spec, cnt_spec],
        out_specs=packed_spec,
        out_shape=jax.ShapeDtypeStruct((NB, BL, 128), jnp.bfloat16),
    )(features, lab3d, csum, ccnt)

    packed = lambda a: a.reshape(N // 128, 128)
    comb2d, probs = pl.pallas_call(
        _pass3_body,
        out_shape=[
            jax.ShapeDtypeStruct((N // 128, 128), jnp.float32),
            jax.ShapeDtypeStruct((N // 128, 128), jnp.float32),
        ],
    )(packed(ms_flat), packed(gn2_flat), packed(dm))

    return comb2d.reshape(N), probs.reshape(N)


# speedup vs baseline: 2.8414x; 1.1799x over previous
import jax
import jax.numpy as jnp
from jax import lax
from jax.experimental import pallas as pl
from jax.experimental.pallas import tpu as pltpu
from jax.experimental.pallas import tpu_sc as plsc

N = 320000
D = 128
CP = 128
B = 16000
NB = N // B
BL = B // 128
BIG = 1e9

SC_NC = 2
SC_NS = 16
SC_NW = SC_NC * SC_NS
RW = N // SC_NW
RCH = 400
NCH = RW // RCH


def _sc_body(g_hbm, gn2_hbm, gb0, gb1, gn2v, sem0, sem1):
    wid = lax.axis_index("s") * SC_NC + lax.axis_index("c")
    base = wid * RW
    gbufs = (gb0, gb1)
    sems = (sem0, sem1)
    lane = lax.iota(jnp.int32, 16)

    def start(ci, b):
        src = pl.ds(base + ci * RCH, RCH)
        pltpu.make_async_copy(g_hbm.at[src, :], gbufs[b], sems[b]).start()

    def wait(b):
        src = pl.ds(base, RCH)
        pltpu.make_async_copy(g_hbm.at[src, :], gbufs[b], sems[b]).wait()

    def gat(x, idx):
        return x.at[idx].get(mode='promise_in_bounds')

    def merge(u, v, s):
        h = s // 2
        uf = u + gat(u, lane ^ h)
        vf = v + gat(v, lane ^ h)
        return jnp.where((lane & h) == 0, uf, gat(vf, lane ^ h))

    def tree(vecs):
        s = 16
        while len(vecs) > 1:
            vecs = [merge(vecs[2 * i], vecs[2 * i + 1], s)
                    for i in range(len(vecs) // 2)]
            s //= 2
        return vecs[0]

    bitrev = (0, 8, 4, 12, 2, 10, 6, 14, 1, 9, 5, 13, 3, 11, 7, 15)

    def compute(ci, b):
        gb = gbufs[b]

        def group(r16, _):
            gvecs = []
            for p in range(16):
                r = r16 * 16 + bitrev[p]
                v = gb[r, pl.ds(0, 16)]
                gacc = v * v
                for j in range(1, 8):
                    v = gb[r, pl.ds(j * 16, 16)]
                    gacc = gacc + v * v
                gvecs.append(gacc)
            gn2v[pl.ds(ci * RCH + r16 * 16, 16)] = tree(gvecs)
            return 0

        lax.fori_loop(0, RCH // 16, group, 0, unroll=False)

    start(0, 0)

    def chunk_pair(ci2, _):
        for b in range(2):
            ci = ci2 * 2 + b

            @pl.when(ci < NCH)
            def _():
                wait(b)

                @pl.when(ci + 1 < NCH)
                def _():
                    start(ci + 1, 1 - b)

                compute(ci, b)
        return 0

    lax.fori_loop(0, (NCH + 1) // 2, chunk_pair, 0, unroll=False)
    pltpu.sync_copy(gn2v, gn2_hbm.at[pl.ds(base, RW)])


def _sc_call(gradients):
    mesh = plsc.VectorSubcoreMesh(core_axis_name="c", subcore_axis_name="s")
    return pl.kernel(
        _sc_body,
        out_type=jax.ShapeDtypeStruct((N,), jnp.float32),
        mesh=mesh,
        scratch_types=[
            pltpu.VMEM((RCH, D), jnp.float32),
            pltpu.VMEM((RCH, D), jnp.float32),
            pltpu.VMEM((RW,), jnp.float32),
            pltpu.SemaphoreType.DMA,
            pltpu.SemaphoreType.DMA,
        ],
    )(gradients)


def _onehot_t_bf16(lab_row):
    ids = jax.lax.broadcasted_iota(jnp.int32, (CP, 1), 0).astype(jnp.bfloat16)
    return jnp.where(lab_row == ids, jnp.bfloat16(1), jnp.bfloat16(0))


def _row_sums_packed(x_bf, ones_col):
    col = jax.lax.dot_general(
        x_bf, ones_col, dimension_numbers=(((1,), (0,)), ((), ())),
        preferred_element_type=jnp.float32)
    return col.astype(jnp.bfloat16).reshape(BL, 128)


def _pass1_body(f_ref, old_ref, lab_ref, ms_ref, csum_ref, ccnt_ref):
    i = pl.program_id(0)
    f = f_ref[...]
    old = old_ref[...]
    lab = lab_ref[0]

    ones_col = jnp.ones((D, 1), dtype=jnp.bfloat16)
    diff = f - old
    ms_ref[...] = _row_sums_packed((diff * diff).astype(jnp.bfloat16),
                                   ones_col)[None]

    oht = _onehot_t_bf16(lab)
    csum_p = jax.lax.dot_general(
        oht, f.astype(jnp.bfloat16),
        dimension_numbers=(((1,), (0,)), ((), ())),
        preferred_element_type=jnp.float32)
    ones_b = jnp.ones((B, 1), dtype=jnp.bfloat16)
    ccnt_p = jax.lax.dot_general(
        oht, ones_b,
        dimension_numbers=(((1,), (0,)), ((), ())),
        preferred_element_type=jnp.float32)

    @pl.when(i == 0)
    def _():
        csum_ref[...] = jnp.zeros_like(csum_ref)
        ccnt_ref[...] = jnp.zeros_like(ccnt_ref)

    csum_ref[...] += csum_p
    ccnt_ref[...] += ccnt_p


def _pass2_body(f_ref, lab_ref, csum_ref, ccnt_ref, dm_ref):
    f = f_ref[...]
    lab = lab_ref[0]
    cnt = ccnt_ref[...]
    inv = 1.0 / jnp.maximum(cnt, 1.0)
    centroids = csum_ref[...] * inv
    pen = jnp.where(cnt > 1.0, 0.0, -BIG).astype(jnp.bfloat16)

    oht = _onehot_t_bf16(lab)
    c_rows = jax.lax.dot_general(
        oht, centroids.astype(jnp.bfloat16),
        dimension_numbers=(((0,), (0,)), ((), ())),
        preferred_element_type=jnp.float32)
    pen_col = jax.lax.dot_general(
        oht, pen,
        dimension_numbers=(((0,), (0,)), ((), ())),
        preferred_element_type=jnp.float32)

    ones_col = jnp.ones((D, 1), dtype=jnp.bfloat16)
    dd = f - c_rows
    dist2_col = jax.lax.dot_general(
        (dd * dd).astype(jnp.bfloat16), ones_col,
        dimension_numbers=(((1,), (0,)), ((), ())),
        preferred_element_type=jnp.float32)
    dm_ref[...] = ((dist2_col + pen_col)
                   .astype(jnp.bfloat16).reshape(BL, 128)[None])


def _pass3_body(ms_ref, gn2_ref, dm_ref, comb_ref, p_ref):
    ls = 1.0 / (1.0 + ms_ref[...].astype(jnp.float32) * (1.0 / D))
    gs = 1.0 / (1.0 + jnp.exp(-jnp.sqrt(gn2_ref[...])))
    dm = dm_ref[...].astype(jnp.float32)
    typ = jnp.where(dm < 0.0, 1.0, 1.0 / (1.0 + jnp.sqrt(jnp.abs(dm))))
    comb = 0.3 * ls + 0.5 * gs + 0.2 * typ
    comb_ref[...] = comb
    m = jnp.max(comb)
    e = jnp.exp(comb - m)
    p_ref[...] = e * (1.0 / jnp.sum(e))


def kernel(features, labels, gradients, old_features):
    lab3d = labels.astype(jnp.int32).astype(jnp.bfloat16).reshape(NB, 1, B)

    gn2_flat = _sc_call(gradients)

    row_spec = pl.BlockSpec((B, D), lambda i: (i, 0))
    lab_spec = pl.BlockSpec((1, 1, B), lambda i: (i, 0, 0))
    packed_spec = pl.BlockSpec((1, BL, 128), lambda i: (i, 0, 0))
    acc_spec = pl.BlockSpec((CP, D), lambda i: (0, 0))
    cnt_spec = pl.BlockSpec((CP, 1), lambda i: (0, 0))

    ms3d, csum, ccnt = pl.pallas_call(
        _pass1_body,
        grid=(NB,),
        in_specs=[row_spec, row_spec, lab_spec],
        out_specs=[packed_spec, acc_spec, cnt_spec],
        out_shape=[
            jax.ShapeDtypeStruct((NB, BL, 128), jnp.bfloat16),
            jax.ShapeDtypeStruct((CP, D), jnp.float32),
            jax.ShapeDtypeStruct((CP, 1), jnp.float32),
        ],
    )(features, old_features, lab3d)

    dm = pl.pallas_call(
        _pass2_body,
        grid=(NB,),
        in_specs=[row_spec, lab_spec, acc_spec, cnt_spec],
        out_specs=packed_spec,
        out_shape=jax.ShapeDtypeStruct((NB, BL, 128), jnp.bfloat16),
    )(features, lab3d, csum, ccnt)

    packed = lambda a: a.reshape(N // 128, 128)
    comb2d, probs = pl.pallas_call(
        _pass3_body,
        out_shape=[
            jax.ShapeDtypeStruct((N // 128, 128), jnp.float32),
            jax.ShapeDtypeStruct((N // 128, 128), jnp.float32),
        ],
    )(packed(ms3d), packed(gn2_flat), packed(dm))

    return comb2d.reshape(N), probs.reshape(N)
